# baseline (speedup 1.0000x reference)
import functools

import jax
import jax.numpy as jnp
from jax import lax
from jax.experimental import pallas as pl
from jax.experimental.pallas import tpu as pltpu
from jax.experimental.pallas import tpu_sc as plsc

N = 10000
D = 128
E = 320000
NC = 2
NS = 16
NW = NC * NS
CHUNK = 128
CB = 8
NB = 10
TCH = CB * NB
EP = NW * TCH * CHUNK
RPS = 624
TAIL = N - NS * RPS


def _spmm_body(sup_hbm, src2d, dst2d, ew2d, z_hbm, out_hbm,
               acc_sh, sidx_blk, didx_blk, ew_blk, rows_v,
               sem_b, sem_r0, sem_r1, sem_w0, sem_w1, sem_z):
    c = lax.axis_index("c")
    s = lax.axis_index("s")
    w = s * NC + c
    ch0 = w * TCH

    def _stage(blk, buf):
        pltpu.async_copy(src2d.at[pl.ds(ch0 + blk * CB, CB)],
                         sidx_blk.at[buf], sem_b)
        pltpu.async_copy(dst2d.at[pl.ds(ch0 + blk * CB, CB)],
                         didx_blk.at[buf], sem_b)
        pltpu.async_copy(ew2d.at[pl.ds(ch0 + blk * CB, CB)],
                         ew_blk.at[buf], sem_b)

    def _stage_wait(buf):
        pltpu.make_async_copy(src2d.at[pl.ds(0, CB)],
                              sidx_blk.at[buf], sem_b).wait()
        pltpu.make_async_copy(dst2d.at[pl.ds(0, CB)],
                              didx_blk.at[buf], sem_b).wait()
        pltpu.make_async_copy(ew2d.at[pl.ds(0, CB)],
                              ew_blk.at[buf], sem_b).wait()

    row0 = s * RPS
    dz = pltpu.async_copy(z_hbm.at[pl.ds(row0, RPS)],
                          acc_sh.at[pl.ds(row0, RPS)], sem_z)

    @pl.when(s == NS - 1)
    def _zero_tail():
        pltpu.async_copy(z_hbm.at[pl.ds(NS * RPS, TAIL)],
                         acc_sh.at[pl.ds(NS * RPS, TAIL)], sem_z)

    _stage(0, 0)
    _stage_wait(0)
    pltpu.async_copy(sup_hbm.at[sidx_blk.at[0, 0]], rows_v.at[0], sem_r0)
    dz.wait()

    @pl.when(s == NS - 1)
    def _zero_tail_wait():
        pltpu.make_async_copy(z_hbm.at[pl.ds(NS * RPS, TAIL)],
                              acc_sh.at[pl.ds(NS * RPS, TAIL)], sem_z).wait()
    plsc.subcore_barrier()

    sem_r = (sem_r0, sem_r1)
    sem_w = (sem_w0, sem_w1)

    def _wait_scatter(buf):
        pltpu.make_async_copy(rows_v.at[buf],
                              acc_sh.at[didx_blk.at[0, 0]], sem_w[buf]).wait()

    def _block(b, carry):
        t0 = b * CB
        bsl = b % 2
        nbsl = 1 - bsl
        for ci in range(CB):
            slot = ci % 2
            other = 1 - slot
            t = t0 + ci

            if ci == 0:
                @pl.when(t >= 1)
                def _retire():
                    _wait_scatter(other)
                @pl.when(b + 1 < NB)
                def _stage_next():
                    _stage(b + 1, nbsl)
            else:
                _wait_scatter(other)

            if ci < CB - 1:
                pltpu.async_copy(sup_hbm.at[sidx_blk.at[bsl, ci + 1]],
                                 rows_v.at[other], sem_r[other])
            else:
                @pl.when(b + 1 < NB)
                def _next_block_gather():
                    _stage_wait(nbsl)
                    pltpu.async_copy(sup_hbm.at[sidx_blk.at[nbsl, 0]],
                                     rows_v.at[other], sem_r[other])

            pltpu.make_async_copy(sup_hbm.at[pl.ds(0, CHUNK)],
                                  rows_v.at[slot], sem_r[slot]).wait()

            def _grp(g, carry2):
                w16 = ew_blk[bsl, ci, pl.ds(g * 16, 16)]
                for j in range(16):
                    r = g * 16 + j
                    wj = w16[j]
                    for cc in range(D // 16):
                        rows_v[slot, r, pl.ds(cc * 16, 16)] = (
                            rows_v[slot, r, pl.ds(cc * 16, 16)] * wj)
                return carry2
            lax.fori_loop(0, CHUNK // 16, _grp, 0)

            pltpu.async_copy(rows_v.at[slot], acc_sh.at[didx_blk.at[bsl, ci]],
                             sem_w[slot], add=True)
        return carry
    lax.fori_loop(0, NB, _block, 0)

    _wait_scatter((TCH - 1) % 2)
    plsc.subcore_barrier()
    pltpu.sync_copy(acc_sh.at[pl.ds(row0, RPS)], out_hbm.at[c, pl.ds(row0, RPS)])

    @pl.when(s == NS - 1)
    def _pub_tail():
        pltpu.sync_copy(acc_sh.at[pl.ds(NS * RPS, TAIL)],
                        out_hbm.at[c, pl.ds(NS * RPS, TAIL)])


_spmm = pl.kernel(
    _spmm_body,
    out_type=jax.ShapeDtypeStruct((NC, N, D), jnp.float32),
    mesh=plsc.VectorSubcoreMesh(core_axis_name="c", subcore_axis_name="s"),
    compiler_params=pltpu.CompilerParams(needs_layout_passes=False),
    scratch_types=[
        pltpu.VMEM_SHARED((N, D), jnp.float32),
        pltpu.VMEM((2, CB, CHUNK), jnp.int32),
        pltpu.VMEM((2, CB, CHUNK), jnp.int32),
        pltpu.VMEM((2, CB, CHUNK), jnp.float32),
        pltpu.VMEM((2, CHUNK, D), jnp.float32),
        pltpu.SemaphoreType.DMA,
        pltpu.SemaphoreType.DMA,
        pltpu.SemaphoreType.DMA,
        pltpu.SemaphoreType.DMA,
        pltpu.SemaphoreType.DMA,
        pltpu.SemaphoreType.DMA,
    ],
)


def _mm0_body(x_ref, w_ref, o_ref):
    o_ref[...] = jnp.dot(x_ref[...], w_ref[...],
                         preferred_element_type=jnp.float32)


def _mid_body(p_ref, b_ref, w_ref, o_ref):
    h = jnp.maximum(p_ref[0] + p_ref[1] + b_ref[...], 0.0)
    o_ref[...] = jnp.dot(h, w_ref[...], preferred_element_type=jnp.float32)


def _fin_body(p_ref, b_ref, o_ref):
    y = p_ref[0] + p_ref[1] + b_ref[...]
    m = jnp.max(y)
    lse = jnp.log(jnp.sum(jnp.exp(y - m))) + m
    o_ref[...] = y - lse


_mm0 = pl.pallas_call(
    _mm0_body, out_shape=jax.ShapeDtypeStruct((N, D), jnp.float32))
_mid = pl.pallas_call(
    _mid_body, out_shape=jax.ShapeDtypeStruct((N, D), jnp.float32))
_fin = pl.pallas_call(
    _fin_body, out_shape=jax.ShapeDtypeStruct((N, D), jnp.float32))


def kernel(features, edge_index, edge_weight,
           W1, W2, W3, W4, W5, W6, W7, W8, W9, W10,
           b1, b2, b3, b4, b5, b6, b7, b8, b9, b10):
    pad = EP - E
    src = jnp.concatenate(
        [edge_index[0].astype(jnp.int32), jnp.zeros((pad,), jnp.int32)]
    ).reshape(EP // CHUNK, CHUNK)
    dst = jnp.concatenate(
        [edge_index[1].astype(jnp.int32), jnp.zeros((pad,), jnp.int32)]
    ).reshape(EP // CHUNK, CHUNK)
    ew = jnp.concatenate(
        [edge_weight.astype(jnp.float32), jnp.zeros((pad,), jnp.float32)]
    ).reshape(EP // CHUNK, CHUNK)
    Ws = [W1, W2, W3, W4, W5, W6, W7, W8, W9, W10]
    bs = [b1, b2, b3, b4, b5, b6, b7, b8, b9, b10]

    zrows = jnp.zeros((N, D), jnp.float32)
    sup = _mm0(features, Ws[0])
    for i in range(1, 10):
        parts = _spmm(sup, src, dst, ew, zrows)
        sup = _mid(parts, bs[i - 1].reshape(1, D), Ws[i])
    parts = _spmm(sup, src, dst, ew, zrows)
    y = _fin(parts, bs[9].reshape(1, D))
    return y.reshape(-1)


# speedup vs baseline: 2.0403x; 2.0403x over previous
import functools

import jax
import jax.numpy as jnp
from jax import lax
from jax.experimental import pallas as pl
from jax.experimental.pallas import tpu as pltpu
from jax.experimental.pallas import tpu_sc as plsc

N = 10000
D = 128
E = 320000
NC = 2
NS = 16
NW = NC * NS
CHUNK = 96
CHN = 105
EPW = CHUNK * CHN
EP = NW * EPW
RPS = 624
TAIL = N - NS * RPS


def _spmm_body(sup_hbm, src_hbm, dst_hbm, ew_hbm, z_hbm, out_hbm,
               acc_sh, sidxf, didx_v, ew_v, rows_v,
               sem_s, sem_r0, sem_r1, sem_r2, sem_m0, sem_m1, sem_m2,
               sem_w0, sem_w1, sem_w2, sem_z):
    c = lax.axis_index("c")
    s = lax.axis_index("s")
    w = s * NC + c
    e0 = w * EPW

    sem_r = (sem_r0, sem_r1, sem_r2)
    sem_m = (sem_m0, sem_m1, sem_m2)
    sem_w = (sem_w0, sem_w1, sem_w2)

    d1 = pltpu.async_copy(src_hbm.at[pl.ds(e0, EPW)], sidxf, sem_s)
    pltpu.async_copy(dst_hbm.at[pl.ds(e0, CHUNK)], didx_v.at[0], sem_m0)
    pltpu.async_copy(ew_hbm.at[pl.ds(e0, CHUNK)], ew_v.at[0], sem_m0)

    row0 = s * RPS
    dz = pltpu.async_copy(z_hbm.at[pl.ds(row0, RPS)],
                          acc_sh.at[pl.ds(row0, RPS)], sem_z)

    @pl.when(s == NS - 1)
    def _zero_tail():
        pltpu.async_copy(z_hbm.at[pl.ds(NS * RPS, TAIL)],
                         acc_sh.at[pl.ds(NS * RPS, TAIL)], sem_z)

    d1.wait()
    pltpu.async_copy(sup_hbm.at[sidxf.at[pl.ds(0, CHUNK)]], rows_v.at[0], sem_r0)
    dz.wait()

    @pl.when(s == NS - 1)
    def _zero_tail_wait():
        pltpu.make_async_copy(z_hbm.at[pl.ds(NS * RPS, TAIL)],
                              acc_sh.at[pl.ds(NS * RPS, TAIL)], sem_z).wait()
    plsc.subcore_barrier()

    def _wait_scatter(buf):
        pltpu.make_async_copy(rows_v.at[buf],
                              acc_sh.at[didx_v.at[buf]], sem_w[buf]).wait()

    def _phase(t, slot):
        nxt = (slot + 1) % 3

        @pl.when(t >= 2)
        def _retire():
            _wait_scatter(nxt)

        @pl.when(t + 1 < CHN)
        def _start_next():
            base = e0 + (t + 1) * CHUNK
            pltpu.async_copy(dst_hbm.at[pl.ds(base, CHUNK)],
                             didx_v.at[nxt], sem_m[nxt])
            pltpu.async_copy(ew_hbm.at[pl.ds(base, CHUNK)],
                             ew_v.at[nxt], sem_m[nxt])
            pltpu.async_copy(sup_hbm.at[sidxf.at[pl.ds((t + 1) * CHUNK, CHUNK)]],
                             rows_v.at[nxt], sem_r[nxt])

        pltpu.make_async_copy(dst_hbm.at[pl.ds(0, CHUNK)],
                              didx_v.at[slot], sem_m[slot]).wait()
        pltpu.make_async_copy(ew_hbm.at[pl.ds(0, CHUNK)],
                              ew_v.at[slot], sem_m[slot]).wait()
        pltpu.make_async_copy(sup_hbm.at[pl.ds(0, CHUNK)],
                              rows_v.at[slot], sem_r[slot]).wait()

        def _grp(g, carry2):
            w16 = ew_v[slot, pl.ds(g * 16, 16)]
            for j in range(16):
                r = g * 16 + j
                wj = w16[j]
                for cc in range(D // 16):
                    rows_v[slot, r, pl.ds(cc * 16, 16)] = (
                        rows_v[slot, r, pl.ds(cc * 16, 16)] * wj)
            return carry2
        lax.fori_loop(0, CHUNK // 16, _grp, 0)

        pltpu.async_copy(rows_v.at[slot], acc_sh.at[didx_v.at[slot]],
                         sem_w[slot], add=True)

    def _tri(i, carry):
        _phase(3 * i, 0)
        _phase(3 * i + 1, 1)
        _phase(3 * i + 2, 2)
        return carry
    lax.fori_loop(0, CHN // 3, _tri, 0)

    _wait_scatter((CHN - 2) % 3)
    _wait_scatter((CHN - 1) % 3)
    plsc.subcore_barrier()
    pltpu.sync_copy(acc_sh.at[pl.ds(row0, RPS)], out_hbm.at[c, pl.ds(row0, RPS)])

    @pl.when(s == NS - 1)
    def _pub_tail():
        pltpu.sync_copy(acc_sh.at[pl.ds(NS * RPS, TAIL)],
                        out_hbm.at[c, pl.ds(NS * RPS, TAIL)])


_spmm = pl.kernel(
    _spmm_body,
    out_type=jax.ShapeDtypeStruct((NC, N, D), jnp.float32),
    mesh=plsc.VectorSubcoreMesh(core_axis_name="c", subcore_axis_name="s"),
    compiler_params=pltpu.CompilerParams(needs_layout_passes=False),
    scratch_types=[
        pltpu.VMEM_SHARED((N, D), jnp.float32),
        pltpu.VMEM((EPW,), jnp.int32),
        pltpu.VMEM((3, CHUNK), jnp.int32),
        pltpu.VMEM((3, CHUNK), jnp.float32),
        pltpu.VMEM((3, CHUNK, D), jnp.float32),
        pltpu.SemaphoreType.DMA,
        pltpu.SemaphoreType.DMA,
        pltpu.SemaphoreType.DMA,
        pltpu.SemaphoreType.DMA,
        pltpu.SemaphoreType.DMA,
        pltpu.SemaphoreType.DMA,
        pltpu.SemaphoreType.DMA,
        pltpu.SemaphoreType.DMA,
        pltpu.SemaphoreType.DMA,
        pltpu.SemaphoreType.DMA,
        pltpu.SemaphoreType.DMA,
    ],
)


def _mm0_body(x_ref, w_ref, o_ref):
    o_ref[...] = jnp.dot(x_ref[...], w_ref[...],
                         preferred_element_type=jnp.float32)


def _mid_body(p_ref, b_ref, w_ref, o_ref):
    h = jnp.maximum(p_ref[0] + p_ref[1] + b_ref[...], 0.0)
    o_ref[...] = jnp.dot(h, w_ref[...], preferred_element_type=jnp.float32)


def _fin_body(p_ref, b_ref, o_ref):
    y = p_ref[0] + p_ref[1] + b_ref[...]
    m = jnp.max(y)
    lse = jnp.log(jnp.sum(jnp.exp(y - m))) + m
    o_ref[...] = y - lse


_mm0 = pl.pallas_call(
    _mm0_body, out_shape=jax.ShapeDtypeStruct((N, D), jnp.float32))
_mid = pl.pallas_call(
    _mid_body, out_shape=jax.ShapeDtypeStruct((N, D), jnp.float32))
_fin = pl.pallas_call(
    _fin_body, out_shape=jax.ShapeDtypeStruct((N, D), jnp.float32))


def kernel(features, edge_index, edge_weight,
           W1, W2, W3, W4, W5, W6, W7, W8, W9, W10,
           b1, b2, b3, b4, b5, b6, b7, b8, b9, b10):
    pad = EP - E
    src = jnp.concatenate(
        [edge_index[0].astype(jnp.int32), jnp.zeros((pad,), jnp.int32)])
    dst = jnp.concatenate(
        [edge_index[1].astype(jnp.int32), jnp.zeros((pad,), jnp.int32)])
    ew = jnp.concatenate(
        [edge_weight.astype(jnp.float32), jnp.zeros((pad,), jnp.float32)])
    Ws = [W1, W2, W3, W4, W5, W6, W7, W8, W9, W10]
    bs = [b1, b2, b3, b4, b5, b6, b7, b8, b9, b10]

    zrows = jnp.zeros((N, D), jnp.float32)
    sup = _mm0(features, Ws[0])
    for i in range(1, 10):
        parts = _spmm(sup, src, dst, ew, zrows)
        sup = _mid(parts, bs[i - 1].reshape(1, D), Ws[i])
    parts = _spmm(sup, src, dst, ew, zrows)
    y = _fin(parts, bs[9].reshape(1, D))
    return y.reshape(-1)


# speedup vs baseline: 3.5221x; 1.7263x over previous
import jax
import jax.numpy as jnp
from jax import lax
from jax.experimental import pallas as pl
from jax.experimental.pallas import tpu as pltpu
from jax.experimental.pallas import tpu_sc as plsc

N = 10000
D = 128
E = 320000
NC = 2
NS = 16
NW = NC * NS
CHUNK = 128
EPW = E // NW
FULL = EPW // CHUNK
TAILE = EPW - FULL * CHUNK
RPS = 624
TAIL = N - NS * RPS


def _spmm_body(sup_hbm, src_hbm, dst_hbm, ew_hbm, z_hbm, out_hbm,
               acc_sh, sidxf, didx_v, ew_v, rows_v,
               sem_s, sem_r0, sem_r1, sem_m0, sem_m1, sem_w0, sem_w1, sem_z):
    c = lax.axis_index("c")
    s = lax.axis_index("s")
    w = s * NC + c
    e0 = w * EPW

    d1 = pltpu.async_copy(src_hbm.at[pl.ds(e0, EPW)], sidxf, sem_s)
    pltpu.async_copy(dst_hbm.at[pl.ds(e0, CHUNK)], didx_v.at[0], sem_m0)
    pltpu.async_copy(ew_hbm.at[pl.ds(e0, CHUNK)], ew_v.at[0], sem_m0)

    row0 = s * RPS
    dz = pltpu.async_copy(z_hbm.at[pl.ds(row0, RPS)],
                          acc_sh.at[pl.ds(row0, RPS)], sem_z)

    @pl.when(s == NS - 1)
    def _zero_tail():
        pltpu.async_copy(z_hbm.at[pl.ds(NS * RPS, TAIL)],
                         acc_sh.at[pl.ds(NS * RPS, TAIL)], sem_z)

    d1.wait()
    pltpu.async_copy(sup_hbm.at[sidxf.at[pl.ds(0, CHUNK)]], rows_v.at[0], sem_r0)
    dz.wait()

    @pl.when(s == NS - 1)
    def _zero_tail_wait():
        pltpu.make_async_copy(z_hbm.at[pl.ds(NS * RPS, TAIL)],
                              acc_sh.at[pl.ds(NS * RPS, TAIL)], sem_z).wait()
    plsc.subcore_barrier()

    sem_r = (sem_r0, sem_r1)
    sem_m = (sem_m0, sem_m1)
    sem_w = (sem_w0, sem_w1)

    def _wait_scatter(buf):
        pltpu.make_async_copy(rows_v.at[buf],
                              acc_sh.at[didx_v.at[buf]], sem_w[buf]).wait()

    def _phase(t, slot, is_tail=False):
        other = 1 - slot

        if not is_tail:
            @pl.when(t >= 1)
            def _retire():
                _wait_scatter(other)

            @pl.when(t + 1 < FULL)
            def _nxt():
                base = e0 + (t + 1) * CHUNK
                pltpu.async_copy(dst_hbm.at[pl.ds(base, CHUNK)],
                                 didx_v.at[other], sem_m[other])
                pltpu.async_copy(ew_hbm.at[pl.ds(base, CHUNK)],
                                 ew_v.at[other], sem_m[other])
                pltpu.async_copy(sup_hbm.at[sidxf.at[pl.ds((t + 1) * CHUNK, CHUNK)]],
                                 rows_v.at[other], sem_r[other])

            @pl.when(t + 1 == FULL)
            def _nxt_tail():
                base = e0 + FULL * CHUNK
                pltpu.async_copy(dst_hbm.at[pl.ds(base, TAILE)],
                                 didx_v.at[other, pl.ds(0, TAILE)], sem_m[other])
                pltpu.async_copy(ew_hbm.at[pl.ds(base, TAILE)],
                                 ew_v.at[other, pl.ds(0, TAILE)], sem_m[other])
                for q in range(TAILE // 16, CHUNK // 16):
                    ew_v[other, pl.ds(q * 16, 16)] = jnp.zeros((16,), jnp.float32)
                pltpu.async_copy(sup_hbm.at[sidxf.at[pl.ds(FULL * CHUNK, TAILE)]],
                                 rows_v.at[other, pl.ds(0, TAILE)], sem_r[other])

        nb = TAILE if is_tail else CHUNK
        pltpu.make_async_copy(dst_hbm.at[pl.ds(0, nb)],
                              didx_v.at[slot, pl.ds(0, nb)], sem_m[slot]).wait()
        pltpu.make_async_copy(ew_hbm.at[pl.ds(0, nb)],
                              ew_v.at[slot, pl.ds(0, nb)], sem_m[slot]).wait()
        pltpu.make_async_copy(sup_hbm.at[pl.ds(0, nb)],
                              rows_v.at[slot, pl.ds(0, nb)], sem_r[slot]).wait()

        def _grp(g, carry2):
            w16 = ew_v[slot, pl.ds(g * 16, 16)]
            for j in range(16):
                r = g * 16 + j
                wj = w16[j]
                for cc in range(D // 16):
                    rows_v[slot, r, pl.ds(cc * 16, 16)] = (
                        rows_v[slot, r, pl.ds(cc * 16, 16)] * wj)
            return carry2
        lax.fori_loop(0, CHUNK // 16, _grp, 0)

        pltpu.async_copy(rows_v.at[slot], acc_sh.at[didx_v.at[slot]],
                         sem_w[slot], add=True)

    def _pair(i, carry):
        _phase(2 * i, 0)
        _phase(2 * i + 1, 1)
        return carry
    lax.fori_loop(0, FULL // 2, _pair, 0)
    _phase(FULL, 0, is_tail=True)

    _wait_scatter(1)
    _wait_scatter(0)
    plsc.subcore_barrier()
    pltpu.sync_copy(acc_sh.at[pl.ds(row0, RPS)], out_hbm.at[c, pl.ds(row0, RPS)])

    @pl.when(s == NS - 1)
    def _pub_tail():
        pltpu.sync_copy(acc_sh.at[pl.ds(NS * RPS, TAIL)],
                        out_hbm.at[c, pl.ds(NS * RPS, TAIL)])


_spmm = pl.kernel(
    _spmm_body,
    out_type=jax.ShapeDtypeStruct((NC, N, D), jnp.float32),
    mesh=plsc.VectorSubcoreMesh(core_axis_name="c", subcore_axis_name="s"),
    compiler_params=pltpu.CompilerParams(needs_layout_passes=False),
    scratch_types=[
        pltpu.VMEM_SHARED((N, D), jnp.float32),
        pltpu.VMEM((EPW,), jnp.int32),
        pltpu.VMEM((2, CHUNK), jnp.int32),
        pltpu.VMEM((2, CHUNK), jnp.float32),
        pltpu.VMEM((2, CHUNK, D), jnp.float32),
        pltpu.SemaphoreType.DMA,
        pltpu.SemaphoreType.DMA,
        pltpu.SemaphoreType.DMA,
        pltpu.SemaphoreType.DMA,
        pltpu.SemaphoreType.DMA,
        pltpu.SemaphoreType.DMA,
        pltpu.SemaphoreType.DMA,
        pltpu.SemaphoreType.DMA,
    ],
)


def _mm0_body(x_ref, w_ref, o_ref):
    o_ref[...] = jnp.dot(x_ref[...], w_ref[...],
                         preferred_element_type=jnp.float32)


def _mid_body(p_ref, b_ref, w_ref, o_ref):
    h = jnp.maximum(p_ref[0] + p_ref[1] + b_ref[...], 0.0)
    o_ref[...] = jnp.dot(h, w_ref[...], preferred_element_type=jnp.float32)


def _fin_body(p_ref, b_ref, o_ref):
    y = p_ref[0] + p_ref[1] + b_ref[...]
    m = jnp.max(y)
    lse = jnp.log(jnp.sum(jnp.exp(y - m))) + m
    o_ref[...] = y - lse


_mm0 = pl.pallas_call(
    _mm0_body, out_shape=jax.ShapeDtypeStruct((N, D), jnp.float32))
_mid = pl.pallas_call(
    _mid_body, out_shape=jax.ShapeDtypeStruct((N, D), jnp.float32))
_fin = pl.pallas_call(
    _fin_body, out_shape=jax.ShapeDtypeStruct((N, D), jnp.float32))


def kernel(features, edge_index, edge_weight,
           W1, W2, W3, W4, W5, W6, W7, W8, W9, W10,
           b1, b2, b3, b4, b5, b6, b7, b8, b9, b10):
    src = edge_index[0].astype(jnp.int32)
    dst = edge_index[1].astype(jnp.int32)
    ew = edge_weight.astype(jnp.float32)
    Ws = [W1, W2, W3, W4, W5, W6, W7, W8, W9, W10]
    bs = [b1, b2, b3, b4, b5, b6, b7, b8, b9, b10]

    zrows = jnp.zeros((N, D), jnp.float32)
    sup = _mm0(features, Ws[0])
    for i in range(1, 10):
        parts = _spmm(sup, src, dst, ew, zrows)
        sup = _mid(parts, bs[i - 1].reshape(1, D), Ws[i])
    parts = _spmm(sup, src, dst, ew, zrows)
    y = _fin(parts, bs[9].reshape(1, D))
    return y.reshape(-1)
